# Initial kernel scaffold; baseline (speedup 1.0000x reference)
import jax
import jax.numpy as jnp
from jax.experimental import pallas as pl


def kernel(s_i_batch, actions_batch, lengths, W_action, b_action, W_stop, b_stop, W_start, b_start):
    raise NotImplementedError("write your pallas kernel here")


# speedup vs baseline: 2.7389x; 2.7389x over previous
import functools

import jax
import jax.numpy as jnp
from jax import lax
from jax.experimental import pallas as pl
from jax.experimental.pallas import tpu as pltpu

B = 16
MAX_T = 4096
S = 128
NA = 256
TB = 512
NT = MAX_T // TB


def _body(lens_ref, s_ref, a_ref, w_ref, b_ref, out_ref):
    i = pl.program_id(0)
    j = pl.program_id(1)
    len_i = lens_ref[i]

    @pl.when((i == 0) & (j == 0))
    def _init():
        out_ref[...] = jnp.zeros_like(out_ref)

    @pl.when(j * TB < len_i)
    def _compute():
        x = s_ref[0]
        logits = jnp.dot(x, w_ref[...], preferred_element_type=jnp.float32)
        logits = logits + b_ref[...]
        m = jnp.max(logits, axis=-1, keepdims=True)
        ex = jnp.exp(logits - m)
        lse = m + jnp.log(jnp.sum(ex, axis=-1, keepdims=True))
        a = a_ref[0]
        lane = lax.broadcasted_iota(jnp.int32, (TB, NA), 1)
        taken = jnp.sum(jnp.where(lane == a, logits, 0.0),
                        axis=-1, keepdims=True)
        trow = j * TB + lax.broadcasted_iota(jnp.int32, (TB, 1), 0)
        valid = trow < len_i
        contrib = jnp.sum(jnp.where(valid, taken - lse, 0.0))
        out_ref[...] = out_ref[...] + contrib


def _s_index(i, j, lens):
    len_i = lens[i]
    jcap = jnp.maximum((len_i + TB - 1) // TB - 1, 0)
    return i, jnp.minimum(j, jcap), 0


def _a_index(i, j, lens):
    len_i = lens[i]
    jcap = jnp.maximum((len_i + TB - 1) // TB - 1, 0)
    return i * NT + jnp.minimum(j, jcap), 0, 0


def kernel(s_i_batch, actions_batch, lengths, W_action, b_action,
           W_stop, b_stop, W_start, b_start):
    del W_stop, b_stop, W_start, b_start
    lens = lengths.astype(jnp.int32)
    acts = jnp.reshape(actions_batch.astype(jnp.int32), (B * NT, TB, 1))
    w0 = W_action[:, :NA]
    b0 = jnp.reshape(b_action[:NA], (1, NA))

    grid_spec = pltpu.PrefetchScalarGridSpec(
        num_scalar_prefetch=1,
        grid=(B, NT),
        in_specs=[
            pl.BlockSpec((1, TB, S), _s_index),
            pl.BlockSpec((1, TB, 1), _a_index),
            pl.BlockSpec((S, NA), lambda i, j, lens: (0, 0)),
            pl.BlockSpec((1, NA), lambda i, j, lens: (0, 0)),
        ],
        out_specs=pl.BlockSpec((1, 1), lambda i, j, lens: (0, 0)),
    )
    total = pl.pallas_call(
        _body,
        grid_spec=grid_spec,
        out_shape=jax.ShapeDtypeStruct((1, 1), jnp.float32),
        compiler_params=pltpu.CompilerParams(
            dimension_semantics=("arbitrary", "arbitrary")),
    )(lens, s_i_batch, acts, w0, b0)
    return -total[0, 0]


# speedup vs baseline: 5.2938x; 1.9328x over previous
import functools

import jax
import jax.numpy as jnp
from jax import lax
from jax.experimental import pallas as pl
from jax.experimental.pallas import tpu as pltpu

B = 16
MAX_T = 4096
S = 128
NA = 256
TB = 1024
NT = MAX_T // TB


def _body(lens_ref, s_ref, a_ref, wt_ref, b_ref, out_ref):
    i = pl.program_id(0)
    j = pl.program_id(1)
    len_i = lens_ref[i]

    @pl.when((i == 0) & (j == 0))
    def _init():
        out_ref[...] = jnp.zeros_like(out_ref)

    @pl.when(j * TB < len_i)
    def _compute():
        x = s_ref[0]
        logits = lax.dot_general(wt_ref[...], x,
                                 (((1,), (1,)), ((), ())),
                                 preferred_element_type=jnp.float32)
        logits = logits + b_ref[...]
        m = jnp.max(logits, axis=0, keepdims=True)
        ex = jnp.exp(logits - m)
        lse = m + jnp.log(jnp.sum(ex, axis=0, keepdims=True))
        a = a_ref[0]
        row = lax.broadcasted_iota(jnp.int32, (NA, TB), 0)
        taken = jnp.sum(jnp.where(row == a, logits, 0.0),
                        axis=0, keepdims=True)
        tcol = j * TB + lax.broadcasted_iota(jnp.int32, (1, TB), 1)
        valid = tcol < len_i
        contrib = jnp.sum(jnp.where(valid, taken - lse, 0.0))
        out_ref[...] = out_ref[...] + contrib


def _s_index(i, j, lens):
    len_i = lens[i]
    jcap = jnp.maximum((len_i + TB - 1) // TB - 1, 0)
    return i, jnp.minimum(j, jcap), 0


def _a_index(i, j, lens):
    len_i = lens[i]
    jcap = jnp.maximum((len_i + TB - 1) // TB - 1, 0)
    return i * NT + jnp.minimum(j, jcap), 0, 0


def kernel(s_i_batch, actions_batch, lengths, W_action, b_action,
           W_stop, b_stop, W_start, b_start):
    del W_stop, b_stop, W_start, b_start
    lens = lengths.astype(jnp.int32)
    acts = jnp.reshape(actions_batch.astype(jnp.int32), (B * NT, 1, TB))
    wt = jnp.transpose(W_action[:, :NA])
    b0 = jnp.reshape(b_action[:NA], (NA, 1))

    grid_spec = pltpu.PrefetchScalarGridSpec(
        num_scalar_prefetch=1,
        grid=(B, NT),
        in_specs=[
            pl.BlockSpec((1, TB, S), _s_index),
            pl.BlockSpec((1, 1, TB), _a_index),
            pl.BlockSpec((NA, S), lambda i, j, lens: (0, 0)),
            pl.BlockSpec((NA, 1), lambda i, j, lens: (0, 0)),
        ],
        out_specs=pl.BlockSpec((1, 1), lambda i, j, lens: (0, 0)),
    )
    total = pl.pallas_call(
        _body,
        grid_spec=grid_spec,
        out_shape=jax.ShapeDtypeStruct((1, 1), jnp.float32),
        compiler_params=pltpu.CompilerParams(
            dimension_semantics=("arbitrary", "arbitrary")),
    )(lens, s_i_batch, acts, wt, b0)
    return -total[0, 0]
